# baseline (speedup 1.0000x reference)
import functools

import jax
import jax.numpy as jnp
from jax import lax
from jax.experimental import pallas as pl
from jax.experimental.pallas import tpu as pltpu
from jax.experimental.pallas import tpu_sc as plsc

N = 2097152
NW = 32
PER_W = N // NW
C = 16384
NCHUNK = PER_W // C

ONE_M_EPS = 1.0 - float(jnp.finfo(jnp.float32).eps)

_mesh = plsc.VectorSubcoreMesh(core_axis_name="c", subcore_axis_name="s")


@functools.partial(
    pl.kernel,
    mesh=_mesh,
    out_type=jax.ShapeDtypeStruct((N,), jnp.float32),
    scratch_types=[
        pltpu.VMEM((C,), jnp.float32),
        pltpu.VMEM((C,), jnp.float32),
        pltpu.VMEM((C,), jnp.float32),
        pltpu.VMEM((C,), jnp.int32),
        pltpu.VMEM((C,), jnp.float32),
        pltpu.SemaphoreType.DMA,
    ],
)
def _grid_gather(xt_hbm, grid_hbm, out_hbm, xv0, xv1, xv2, idxv, outv, sem):
    wid = lax.axis_index("s") * 2 + lax.axis_index("c")
    base = wid * PER_W

    def chunk_body(ci, carry):
        cbase = base + ci * C
        pltpu.sync_copy(xt_hbm.at[pl.ds(cbase, C)], xv0)
        pltpu.sync_copy(xt_hbm.at[pl.ds(N + cbase, C)], xv1)
        pltpu.sync_copy(xt_hbm.at[pl.ds(2 * N + cbase, C)], xv2)

        def vec_body(i, carry2):
            s = i * 16
            x0 = xv0[pl.ds(s, 16)]
            x1 = xv1[pl.ds(s, 16)]
            x2 = xv2[pl.ds(s, 16)]

            def to_cell(v):
                t = (v + 1.0) * 0.5
                t = jnp.minimum(jnp.maximum(t, 0.0), ONE_M_EPS)
                return (t * 256.0).astype(jnp.int32)

            lin = (to_cell(x0) << 16) | (to_cell(x1) << 8) | to_cell(x2)
            idxv[pl.ds(s, 16)] = lin
            return carry2

        lax.fori_loop(0, C // 16, vec_body, 0, unroll=4)

        pltpu.async_copy(grid_hbm.at[idxv], outv, sem).wait()
        pltpu.sync_copy(outv, out_hbm.at[pl.ds(cbase, C)])
        return carry

    lax.fori_loop(0, NCHUNK, chunk_body, 0)


def kernel(x, grid):
    xt = x.T.reshape(3 * N)
    grid_flat = grid.reshape(-1)
    return _grid_gather(xt, grid_flat)
